# baseline (speedup 1.0000x reference)
import functools

import jax
import jax.numpy as jnp
from jax import lax
from jax.experimental import pallas as pl
from jax.experimental.pallas import tpu as pltpu
from jax.experimental.pallas import tpu_sc as plsc

BATCH = 16384
N_FIELDS = 26
D_EMB = 64
N_ROWS = BATCH * N_FIELDS
D_PAIR = 2 * D_EMB
N_EMB_PAIRS = 500000

_NC = 2
_NS = 16
_NW = _NC * _NS
_ROWS_PER_W = N_ROWS // _NW
_CROWS = 416
_N_CHUNKS = _ROWS_PER_W // _CROWS

_mesh = plsc.VectorSubcoreMesh(core_axis_name="c", subcore_axis_name="s")


@functools.partial(
    pl.kernel,
    mesh=_mesh,
    out_type=jax.ShapeDtypeStruct((N_ROWS, D_PAIR), jnp.float32),
    scratch_types=[
        pltpu.VMEM((_ROWS_PER_W,), jnp.int32),
        pltpu.VMEM((_CROWS, D_PAIR), jnp.float32),
        pltpu.VMEM((_CROWS, D_PAIR), jnp.float32),
        pltpu.SemaphoreType.DMA,
        pltpu.SemaphoreType.DMA,
        pltpu.SemaphoreType.DMA,
        pltpu.SemaphoreType.DMA,
    ],
    compiler_params=pltpu.CompilerParams(use_tc_tiling_on_sc=True),
)
def _gather_sc(idx_hbm, table_hbm, out_hbm, idx_v, rows0, rows1,
               gsem0, gsem1, osem0, osem1):
    wid = lax.axis_index("s") * _NC + lax.axis_index("c")
    base = wid * _ROWS_PER_W
    pltpu.sync_copy(idx_hbm.at[pl.ds(base, _ROWS_PER_W)], idx_v)

    rows = (rows0, rows1)
    gsem = (gsem0, gsem1)
    osem = (osem0, osem1)

    def start_gather(i):
        b = i % 2
        return pltpu.async_copy(
            table_hbm.at[idx_v.at[pl.ds(i * _CROWS, _CROWS)]], rows[b], gsem[b])

    def start_out(i):
        b = i % 2
        return pltpu.async_copy(
            rows[b], out_hbm.at[pl.ds(base + i * _CROWS, _CROWS)], osem[b])

    g = [None] * _N_CHUNKS
    o = [None] * _N_CHUNKS
    g[0] = start_gather(0)
    g[1] = start_gather(1)
    for i in range(_N_CHUNKS):
        g[i].wait()
        o[i] = start_out(i)
        if i + 2 < _N_CHUNKS:
            o[i].wait()
            g[i + 2] = start_gather(i + 2)
    o[_N_CHUNKS - 2].wait()
    o[_N_CHUNKS - 1].wait()


def kernel(idx, table):
    idx = idx.astype(jnp.int32)
    pair_idx = (idx >> 1).reshape(-1)
    pairs = _gather_sc(pair_idx, table.reshape(N_EMB_PAIRS, D_PAIR))
    pairs = pairs.reshape(BATCH, N_FIELDS, 2, D_EMB)
    parity = (idx & 1)[:, :, None].astype(jnp.bool_)
    return jnp.where(parity, pairs[:, :, 1, :], pairs[:, :, 0, :])
